# baseline (speedup 1.0000x reference)
import jax
import jax.numpy as jnp
from jax import lax
from jax.experimental import pallas as pl
from jax.experimental.pallas import tpu as pltpu
from jax.experimental.pallas import tpu_sc as plsc

VOCAB = 100000
EMBED = 128
BATCH = 4096
HIST = 50

NC = 2
NS = 16
NW = NC * NS

ROWS = BATCH * HIST
R_PER_W = ROWS // NW
C = 128
CHUNKS = R_PER_W // C


def _gather_body(table_hbm, idx_hbm, out_hbm, idx_v, buf_a, buf_b,
                 gsem_a, gsem_b, wsem_a, wsem_b):
    wid = lax.axis_index("s") * NC + lax.axis_index("c")
    row_base = wid * R_PER_W

    pltpu.sync_copy(idx_hbm.at[wid], idx_v)

    def fire_gather(buf, sem, j):
        pltpu.async_copy(table_hbm.at[idx_v.at[j]], buf, sem)

    def wait_gather(buf, sem, j):
        pltpu.make_async_copy(table_hbm.at[idx_v.at[j]], buf, sem).wait()

    def fire_write(buf, sem, j):
        pltpu.async_copy(buf, out_hbm.at[pl.ds(row_base + j * C, C)], sem)

    def wait_write(buf, sem, j):
        pltpu.make_async_copy(
            buf, out_hbm.at[pl.ds(row_base + j * C, C)], sem).wait()

    fire_gather(buf_a, gsem_a, 0)
    wait_gather(buf_a, gsem_a, 0)
    fire_write(buf_a, wsem_a, 0)
    fire_gather(buf_b, gsem_b, 1)

    def step(m, carry):
        j1 = 2 * m + 1
        wait_gather(buf_b, gsem_b, j1)
        fire_write(buf_b, wsem_b, j1)
        wait_write(buf_a, wsem_a, j1 - 1)
        fire_gather(buf_a, gsem_a, j1 + 1)

        j2 = 2 * m + 2
        wait_gather(buf_a, gsem_a, j2)
        fire_write(buf_a, wsem_a, j2)
        wait_write(buf_b, wsem_b, j2 - 1)
        fire_gather(buf_b, gsem_b, j2 + 1)
        return carry

    lax.fori_loop(0, CHUNKS // 2 - 1, step, 0)

    j_last = CHUNKS - 1
    wait_gather(buf_b, gsem_b, j_last)
    fire_write(buf_b, wsem_b, j_last)
    wait_write(buf_a, wsem_a, j_last - 1)
    wait_write(buf_b, wsem_b, j_last)


@jax.jit
def _embed(indices, table):
    idx3 = indices.reshape(NW, CHUNKS, C)
    mesh = plsc.VectorSubcoreMesh(
        core_axis_name="c", subcore_axis_name="s", num_cores=NC, num_subcores=NS
    )
    out_flat = pl.kernel(
        _gather_body,
        out_type=jax.ShapeDtypeStruct((ROWS, EMBED), jnp.float32),
        mesh=mesh,
        scratch_types=[
            pltpu.VMEM((CHUNKS, C), jnp.int32),
            pltpu.VMEM((C, EMBED), jnp.float32),
            pltpu.VMEM((C, EMBED), jnp.float32),
            pltpu.SemaphoreType.DMA,
            pltpu.SemaphoreType.DMA,
            pltpu.SemaphoreType.DMA,
            pltpu.SemaphoreType.DMA,
        ],
    )(table, idx3)
    return out_flat.reshape(BATCH, HIST, EMBED)


def kernel(indices, table):
    return _embed(indices, table)


# speedup vs baseline: 1.0688x; 1.0688x over previous
import jax
import jax.numpy as jnp
from jax import lax
from jax.experimental import pallas as pl
from jax.experimental.pallas import tpu as pltpu
from jax.experimental.pallas import tpu_sc as plsc

VOCAB = 100000
EMBED = 128
BATCH = 4096
HIST = 50

NC = 2
NS = 16
NW = NC * NS

ROWS = BATCH * HIST
R_PER_W = ROWS // NW
C = 128
CHUNKS = R_PER_W // C
RING = 6


def _gather_body(table_hbm, idx_hbm, out_hbm, idx_v, *ring):
    bufs = ring[:RING]
    gsems = ring[RING:2 * RING]
    wsems = ring[2 * RING:]

    wid = lax.axis_index("s") * NC + lax.axis_index("c")
    row_base = wid * R_PER_W

    pltpu.sync_copy(idx_hbm.at[wid], idx_v)

    def fire_gather(r, j):
        pltpu.async_copy(table_hbm.at[idx_v.at[j]], bufs[r], gsems[r])

    def wait_gather(r, j):
        pltpu.make_async_copy(table_hbm.at[idx_v.at[j]], bufs[r],
                              gsems[r]).wait()

    def fire_write(r, j):
        pltpu.async_copy(
            bufs[r], out_hbm.at[pl.ds(row_base + j * C, C)], wsems[r])

    def wait_write(r, j):
        pltpu.make_async_copy(
            bufs[r], out_hbm.at[pl.ds(row_base + j * C, C)], wsems[r]).wait()

    def retire(j, r):
        wait_gather(r, j)
        fire_write(r, j)
        h = RING // 2
        wait_write((r + h) % RING, j - h)
        fire_gather((r + h) % RING, j + h)

    for k in range(RING):
        fire_gather(k, k)
    for j in range(RING // 2):
        wait_gather(j, j)
        fire_write(j, j)

    h = RING // 2
    loop_start = ((h + RING - 1) // RING + 1) * RING
    for j in range(h, loop_start):
        retire(j, j % RING)

    n_steady = (CHUNKS - h) - loop_start
    n_iter = n_steady // RING

    def step(m, carry):
        j0 = loop_start + m * RING
        for r in range(RING):
            retire(j0 + r, r)
        return carry

    lax.fori_loop(0, n_iter, step, 0)

    for j in range(loop_start + n_iter * RING, CHUNKS - h):
        retire(j, j % RING)
    for j in range(CHUNKS - h, CHUNKS):
        wait_gather(j % RING, j)
        fire_write(j % RING, j)
        wait_write((j + h) % RING, j - h)
    for j in range(CHUNKS - h, CHUNKS):
        wait_write(j % RING, j)


@jax.jit
def _embed(indices, table):
    idx3 = indices.reshape(NW, CHUNKS, C)
    mesh = plsc.VectorSubcoreMesh(
        core_axis_name="c", subcore_axis_name="s", num_cores=NC, num_subcores=NS
    )
    out_flat = pl.kernel(
        _gather_body,
        out_type=jax.ShapeDtypeStruct((ROWS, EMBED), jnp.float32),
        mesh=mesh,
        scratch_types=(
            [pltpu.VMEM((CHUNKS, C), jnp.int32)]
            + [pltpu.VMEM((C, EMBED), jnp.float32) for _ in range(RING)]
            + [pltpu.SemaphoreType.DMA for _ in range(2 * RING)]
        ),
    )(table, idx3)
    return out_flat.reshape(BATCH, HIST, EMBED)


def kernel(indices, table):
    return _embed(indices, table)


# speedup vs baseline: 1.9055x; 1.7829x over previous
import jax
import jax.numpy as jnp
from jax import lax
from jax.experimental import pallas as pl
from jax.experimental.pallas import tpu as pltpu
from jax.experimental.pallas import tpu_sc as plsc

VOCAB = 100000
EMBED = 128
BATCH = 4096
HIST = 50

NC = 2
NS = 16
NW = NC * NS

B_PER_W = BATCH // NW
CHUNKS = B_PER_W
RING = 12
H = RING // 2


def _gather_body(table_hbm, idx_hbm, out_hbm, idx_v, *ring):
    bufs = ring[:RING]
    gsems = ring[RING:2 * RING]
    wsems = ring[2 * RING:]

    wid = lax.axis_index("s") * NC + lax.axis_index("c")
    batch_base = wid * B_PER_W

    pltpu.sync_copy(idx_hbm.at[pl.ds(batch_base, B_PER_W)], idx_v)

    def fire_gather(r, j):
        pltpu.async_copy(table_hbm.at[idx_v.at[j]], bufs[r], gsems[r])

    def wait_gather(r, j):
        pltpu.make_async_copy(table_hbm.at[idx_v.at[j]], bufs[r],
                              gsems[r]).wait()

    def fire_write(r, j):
        pltpu.async_copy(bufs[r], out_hbm.at[batch_base + j], wsems[r])

    def wait_write(r, j):
        pltpu.make_async_copy(
            bufs[r], out_hbm.at[batch_base + j], wsems[r]).wait()

    def retire(j, r):
        wait_gather(r, j)
        fire_write(r, j)
        wait_write((r + H) % RING, j - H)
        fire_gather((r + H) % RING, j + H)

    for k in range(RING):
        fire_gather(k, k)
    for j in range(H):
        wait_gather(j, j)
        fire_write(j, j)

    loop_start = ((H + RING - 1) // RING + 1) * RING
    for j in range(H, loop_start):
        retire(j, j % RING)

    n_steady = (CHUNKS - H) - loop_start
    n_iter = n_steady // RING

    def step(m, carry):
        j0 = loop_start + m * RING
        for r in range(RING):
            retire(j0 + r, r)
        return carry

    lax.fori_loop(0, n_iter, step, 0)

    for j in range(loop_start + n_iter * RING, CHUNKS - H):
        retire(j, j % RING)
    for j in range(CHUNKS - H, CHUNKS):
        wait_gather(j % RING, j)
        fire_write(j % RING, j)
        wait_write((j + H) % RING, j - H)
    for j in range(CHUNKS - H, CHUNKS):
        wait_write(j % RING, j)


@jax.jit
def _embed(indices, table):
    mesh = plsc.VectorSubcoreMesh(
        core_axis_name="c", subcore_axis_name="s", num_cores=NC, num_subcores=NS
    )
    return pl.kernel(
        _gather_body,
        out_type=jax.ShapeDtypeStruct((BATCH, HIST, EMBED), jnp.float32),
        mesh=mesh,
        scratch_types=(
            [pltpu.VMEM((B_PER_W, HIST), jnp.int32)]
            + [pltpu.VMEM((HIST, EMBED), jnp.float32) for _ in range(RING)]
            + [pltpu.SemaphoreType.DMA for _ in range(2 * RING)]
        ),
    )(table, indices)


def kernel(indices, table):
    return _embed(indices, table)
